# Initial kernel scaffold; baseline (speedup 1.0000x reference)
import jax
import jax.numpy as jnp
from jax.experimental import pallas as pl


def kernel(video_ids, categories, tags, durations, timestamps, video_table, category_table, tag_table, duration_table, time_table, W1, b1, W2, b2, gamma, beta):
    raise NotImplementedError("write your pallas kernel here")


# speedup vs baseline: 2.1452x; 2.1452x over previous
import functools

import jax
import jax.numpy as jnp
from jax import lax
from jax.experimental import pallas as pl
from jax.experimental.pallas import tpu as pltpu
from jax.experimental.pallas import tpu_sc as plsc

_EMB = 64
_NC = 2
_NS = 16
_NW = _NC * _NS
_G = 128


def _sc_gather_all(idx2d, tables, n_tok):
    ng_total = n_tok // _G
    ng = ng_total // _NW
    tpw = n_tok // _NW

    mesh = plsc.VectorSubcoreMesh(core_axis_name="c", subcore_axis_name="s",
                                  num_cores=_NC, num_subcores=_NS)

    @functools.partial(
        pl.kernel,
        out_type=tuple(jax.ShapeDtypeStruct((n_tok, _EMB), jnp.float32)
                       for _ in range(5)),
        mesh=mesh,
        scratch_types=[
            pltpu.VMEM((5, ng, _G), jnp.int32),
            pltpu.VMEM((5, _G, _EMB), jnp.float32),
            pltpu.SemaphoreType.DMA,
        ],
        compiler_params=pltpu.CompilerParams(use_tc_tiling_on_sc=False),
    )
    def k(idx_hbm, vt, ct, tt, dt, tit,
          o_v, o_c, o_t, o_d, o_ti, idx_v, rows_v, sem):
        wid = lax.axis_index("s") * _NC + lax.axis_index("c")
        tbase = wid * tpw
        tabs = (vt, ct, tt, dt, tit)
        outs = (o_v, o_c, o_t, o_d, o_ti)
        for t in range(5):
            pltpu.sync_copy(idx_hbm.at[t, wid], idx_v.at[t])

        def grp(g, carry):
            cps = []
            for t in range(5):
                cps.append(pltpu.async_copy(
                    tabs[t].at[idx_v.at[t, g]], rows_v.at[t], sem))
            for t in range(5):
                cps[t].wait()
            for t in range(5):
                pltpu.sync_copy(rows_v.at[t],
                                outs[t].at[pl.ds(tbase + g * _G, _G)])
            return carry

        lax.fori_loop(0, ng, grp, 0)

    return k(idx2d, *tables)


def _tc_mlp(rows, W1, b1, W2, b2, gamma, beta, n_tok, blk):
    grid = (n_tok // blk,)

    def body(v, c, t, d, ti, w1, b1r, w2, b2r, gm, bt, o):
        acc = jnp.dot(v[...], w1[0:64, :], preferred_element_type=jnp.float32)
        acc = acc + jnp.dot(c[...], w1[64:128, :], preferred_element_type=jnp.float32)
        acc = acc + jnp.dot(t[...], w1[128:192, :], preferred_element_type=jnp.float32)
        acc = acc + jnp.dot(d[...], w1[192:256, :], preferred_element_type=jnp.float32)
        acc = acc + jnp.dot(ti[...], w1[256:320, :], preferred_element_type=jnp.float32)
        h = jnp.maximum(acc + b1r[...], 0.0)
        h2 = jnp.dot(h, w2[...], preferred_element_type=jnp.float32) + b2r[...]
        mu = jnp.mean(h2, axis=-1, keepdims=True)
        var = jnp.mean((h2 - mu) ** 2, axis=-1, keepdims=True)
        o[...] = (h2 - mu) / jnp.sqrt(var + 1e-3) * gm[...] + bt[...]

    tok_spec = pl.BlockSpec((blk, _EMB), lambda i: (i, 0))
    full = lambda shape: pl.BlockSpec(shape, lambda i: tuple(0 for _ in shape))
    return pl.pallas_call(
        body,
        grid=grid,
        in_specs=[tok_spec] * 5 + [
            full((5 * _EMB, _EMB)), full((1, _EMB)),
            full((_EMB, _EMB)), full((1, _EMB)),
            full((1, _EMB)), full((1, _EMB)),
        ],
        out_specs=tok_spec,
        out_shape=jax.ShapeDtypeStruct((n_tok, _EMB), jnp.float32),
    )(*rows, W1, b1.reshape(1, _EMB), W2, b2.reshape(1, _EMB),
      gamma.reshape(1, _EMB), beta.reshape(1, _EMB))


def kernel(video_ids, categories, tags, durations, timestamps,
           video_table, category_table, tag_table, duration_table, time_table,
           W1, b1, W2, b2, gamma, beta):
    B, L = video_ids.shape
    n_tok = B * L
    dur_buckets = (durations / 300.0 * 100.0).astype(jnp.int32)
    time_buckets = (timestamps % 168).astype(jnp.int32)
    idx2d = jnp.stack([
        video_ids.reshape(-1).astype(jnp.int32),
        categories.reshape(-1).astype(jnp.int32),
        tags.reshape(-1).astype(jnp.int32),
        dur_buckets.reshape(-1),
        time_buckets.reshape(-1),
    ]).reshape(5, _NW, n_tok // (_NW * _G), _G)

    rows = _sc_gather_all(
        idx2d,
        [video_table, category_table, tag_table, duration_table, time_table],
        n_tok)
    out = _tc_mlp(rows, W1, b1, W2, b2, gamma, beta, n_tok, blk=2048)
    return out.reshape(B, L, _EMB)


# speedup vs baseline: 3.1275x; 1.4579x over previous
import functools

import jax
import jax.numpy as jnp
from jax import lax
from jax.experimental import pallas as pl
from jax.experimental.pallas import tpu as pltpu
from jax.experimental.pallas import tpu_sc as plsc

_EMB = 64
_NC = 2
_NS = 16
_NW = _NC * _NS
_G = 128


def _prep_table(table):
    V = table.shape[0]
    tT = table.T
    blk = min(V, 2048)
    grid = ((V + blk - 1) // blk,)

    def body(x, o):
        y = x[...].T
        z = jnp.concatenate([y, jnp.zeros((blk, _EMB), jnp.float32)], axis=1)
        o[...] = z.reshape(blk * 2 * _EMB)

    out = pl.pallas_call(
        body,
        grid=grid,
        in_specs=[pl.BlockSpec((_EMB, blk), lambda i: (0, i))],
        out_specs=pl.BlockSpec((blk * 2 * _EMB,), lambda i: (i,)),
        out_shape=jax.ShapeDtypeStruct((V * 2 * _EMB,), jnp.float32),
    )(tT)
    return out.reshape(2 * V, _EMB)


def _sc_gather_all(idx2d, tables, n_tok):
    ng = n_tok // (_NW * _G)
    tpw = n_tok // _NW

    mesh = plsc.VectorSubcoreMesh(core_axis_name="c", subcore_axis_name="s",
                                  num_cores=_NC, num_subcores=_NS)

    @functools.partial(
        pl.kernel,
        out_type=tuple(jax.ShapeDtypeStruct((n_tok, _EMB), jnp.float32)
                       for _ in range(5)),
        mesh=mesh,
        scratch_types=[
            pltpu.VMEM((5, ng, _G), jnp.int32),
            pltpu.VMEM((5, _G, _EMB), jnp.float32),
            pltpu.SemaphoreType.DMA,
        ],
        compiler_params=pltpu.CompilerParams(use_tc_tiling_on_sc=False),
    )
    def k(idx_hbm, vt, ct, tt, dt, tit,
          o_v, o_c, o_t, o_d, o_ti, idx_v, rows_v, sem):
        wid = lax.axis_index("s") * _NC + lax.axis_index("c")
        tbase = wid * tpw
        tabs = (vt, ct, tt, dt, tit)
        outs = (o_v, o_c, o_t, o_d, o_ti)
        for t in range(5):
            pltpu.sync_copy(idx_hbm.at[t, wid], idx_v.at[t])

        def grp(g, carry):
            cps = []
            for t in range(5):
                cps.append(pltpu.async_copy(
                    tabs[t].at[idx_v.at[t, g]], rows_v.at[t], sem))
            for t in range(5):
                cps[t].wait()
            for t in range(5):
                pltpu.sync_copy(rows_v.at[t],
                                outs[t].at[pl.ds(tbase + g * _G, _G)])
            return carry

        lax.fori_loop(0, ng, grp, 0)

    return k(idx2d, *tables)


def _tc_mlp(rows128, W1, b1, W2, b2, gamma, beta, n_half, blk):
    grid = (n_half // blk,)

    def half(xs, w1, b1r, w2, b2r, gm, bt, lo):
        acc = jnp.dot(xs[0], w1[0:64, :], preferred_element_type=jnp.float32)
        for t in range(1, 5):
            acc = acc + jnp.dot(xs[t], w1[64 * t:64 * (t + 1), :],
                                preferred_element_type=jnp.float32)
        h = jnp.maximum(acc + b1r, 0.0)
        h2 = jnp.dot(h, w2, preferred_element_type=jnp.float32) + b2r
        mu = jnp.mean(h2, axis=-1, keepdims=True)
        var = jnp.mean((h2 - mu) ** 2, axis=-1, keepdims=True)
        return (h2 - mu) / jnp.sqrt(var + 1e-3) * gm + bt

    def body(v, c, t, d, ti, w1, b1r, w2, b2r, gm, bt, o):
        ins = (v[...], c[...], t[...], d[...], ti[...])
        args = (w1[...], b1r[...], w2[...], b2r[...], gm[...], bt[...])
        re = half(tuple(x[:, 0:64] for x in ins), *args, 0)
        ro = half(tuple(x[:, 64:128] for x in ins), *args, 1)
        o[...] = jnp.concatenate([re, ro], axis=1)

    tok_spec = pl.BlockSpec((blk, 2 * _EMB), lambda i: (i, 0))
    full = lambda shape: pl.BlockSpec(shape, lambda i: tuple(0 for _ in shape))
    return pl.pallas_call(
        body,
        grid=grid,
        in_specs=[tok_spec] * 5 + [
            full((5 * _EMB, _EMB)), full((1, _EMB)),
            full((_EMB, _EMB)), full((1, _EMB)),
            full((1, _EMB)), full((1, _EMB)),
        ],
        out_specs=tok_spec,
        out_shape=jax.ShapeDtypeStruct((n_half, 2 * _EMB), jnp.float32),
    )(*rows128, W1, b1.reshape(1, _EMB), W2, b2.reshape(1, _EMB),
      gamma.reshape(1, _EMB), beta.reshape(1, _EMB))


def kernel(video_ids, categories, tags, durations, timestamps,
           video_table, category_table, tag_table, duration_table, time_table,
           W1, b1, W2, b2, gamma, beta):
    B, L = video_ids.shape
    n_tok = B * L
    dur_buckets = (durations / 300.0 * 100.0).astype(jnp.int32)
    time_buckets = (timestamps % 168).astype(jnp.int32)
    idx2d = (jnp.stack([
        video_ids.reshape(-1).astype(jnp.int32),
        categories.reshape(-1).astype(jnp.int32),
        tags.reshape(-1).astype(jnp.int32),
        dur_buckets.reshape(-1),
        time_buckets.reshape(-1),
    ]) * 2).reshape(5, _NW, n_tok // (_NW * _G), _G)

    tables = [_prep_table(t) for t in
              (video_table, category_table, tag_table,
               duration_table, time_table)]
    rows = _sc_gather_all(idx2d, tables, n_tok)
    rows128 = [r.reshape(n_tok // 2, 2 * _EMB) for r in rows]
    out = _tc_mlp(rows128, W1, b1, W2, b2, gamma, beta, n_tok // 2, blk=1024)
    return out.reshape(B, L, _EMB)


# speedup vs baseline: 3.7584x; 1.2017x over previous
import functools

import jax
import jax.numpy as jnp
from jax import lax
from jax.experimental import pallas as pl
from jax.experimental.pallas import tpu as pltpu
from jax.experimental.pallas import tpu_sc as plsc

_EMB = 64
_NC = 2
_NS = 16
_NW = _NC * _NS
_G = 128


def _prep_table(table, blk):
    V = table.shape[0]
    tT = table.T
    blk = min(V, blk)
    grid = ((V + blk - 1) // blk,)

    def body(x, o):
        y = x[...].T
        z = jnp.concatenate([y, jnp.zeros((blk, _EMB), jnp.float32)], axis=1)
        o[...] = z.reshape(blk * 2 * _EMB)

    out = pl.pallas_call(
        body,
        grid=grid,
        in_specs=[pl.BlockSpec((_EMB, blk), lambda i: (0, i))],
        out_specs=pl.BlockSpec((blk * 2 * _EMB,), lambda i: (i,)),
        out_shape=jax.ShapeDtypeStruct((V * 2 * _EMB,), jnp.float32),
    )(tT)
    return out.reshape(2 * V, _EMB)


def _sc_gather(idx2d, tables, n_tok):
    nt = len(tables)
    ng = n_tok // (_NW * _G)
    tpw = n_tok // _NW

    mesh = plsc.VectorSubcoreMesh(core_axis_name="c", subcore_axis_name="s",
                                  num_cores=_NC, num_subcores=_NS)

    @functools.partial(
        pl.kernel,
        out_type=tuple(jax.ShapeDtypeStruct((n_tok, _EMB), jnp.float32)
                       for _ in range(nt)),
        mesh=mesh,
        scratch_types=[
            pltpu.VMEM((nt, ng, _G), jnp.int32),
            pltpu.VMEM((nt, _G, _EMB), jnp.float32),
            pltpu.VMEM((nt, _G, _EMB), jnp.float32),
            pltpu.SemaphoreType.DMA,
            pltpu.SemaphoreType.DMA,
        ],
        compiler_params=pltpu.CompilerParams(use_tc_tiling_on_sc=False),
    )
    def k(idx_hbm, *rest):
        tabs = rest[:nt]
        outs = rest[nt:2 * nt]
        idx_v, rows_a, rows_b, sem_a, sem_b = rest[2 * nt:]
        wid = lax.axis_index("s") * _NC + lax.axis_index("c")
        tbase = wid * tpw
        for t in range(nt):
            pltpu.sync_copy(idx_hbm.at[t, wid], idx_v.at[t])

        def fire(g, buf, sem):
            return [pltpu.async_copy(tabs[t].at[idx_v.at[t, g]],
                                     buf.at[t], sem)
                    for t in range(nt)]

        fire(0, rows_a, sem_a)

        def pair2(i, carry):
            g0 = i * 2

            cps_b = fire(g0 + 1, rows_b, sem_b)
            for t in range(nt):
                pltpu.make_async_copy(tabs[t].at[idx_v.at[t, g0]],
                                      rows_a.at[t], sem_a).wait()
            for t in range(nt):
                pltpu.sync_copy(rows_a.at[t],
                                outs[t].at[pl.ds(tbase + g0 * _G, _G)])

            @pl.when(i < ng // 2 - 1)
            def _():
                fire(g0 + 2, rows_a, sem_a)

            for cp in cps_b:
                cp.wait()
            for t in range(nt):
                pltpu.sync_copy(rows_b.at[t],
                                outs[t].at[pl.ds(tbase + (g0 + 1) * _G, _G)])
            return carry

        lax.fori_loop(0, ng // 2, pair2, 0)

    return k(idx2d, *tables)


def _tc_mlp(rows128, W1, b1, W2, b2, gamma, beta, n_half, blk):
    grid = (n_half // blk,)

    def half(xs, w1, b1r, w2, b2r, gm, bt):
        acc = jnp.dot(xs[0], w1[0:64, :], preferred_element_type=jnp.float32)
        for t in range(1, 5):
            acc = acc + jnp.dot(xs[t], w1[64 * t:64 * (t + 1), :],
                                preferred_element_type=jnp.float32)
        h = jnp.maximum(acc + b1r, 0.0)
        h2 = jnp.dot(h, w2, preferred_element_type=jnp.float32) + b2r
        mu = jnp.mean(h2, axis=-1, keepdims=True)
        var = jnp.mean((h2 - mu) ** 2, axis=-1, keepdims=True)
        return (h2 - mu) / jnp.sqrt(var + 1e-3) * gm + bt

    def body(v, c, t, d, ti, w1, b1r, w2, b2r, gm, bt, o):
        ins = (v[...], c[...], t[...], d[...], ti[...])
        args = (w1[...], b1r[...], w2[...], b2r[...], gm[...], bt[...])
        re = half(tuple(x[:, 0:64] for x in ins), *args)
        ro = half(tuple(x[:, 64:128] for x in ins), *args)
        o[...] = jnp.concatenate([re, ro], axis=1)

    tok_spec = pl.BlockSpec((blk, 2 * _EMB), lambda i: (i, 0))
    full = lambda shape: pl.BlockSpec(shape, lambda i: tuple(0 for _ in shape))
    return pl.pallas_call(
        body,
        grid=grid,
        in_specs=[tok_spec] * 5 + [
            full((5 * _EMB, _EMB)), full((1, _EMB)),
            full((_EMB, _EMB)), full((1, _EMB)),
            full((1, _EMB)), full((1, _EMB)),
        ],
        out_specs=tok_spec,
        out_shape=jax.ShapeDtypeStruct((n_half, 2 * _EMB), jnp.float32),
    )(*rows128, W1, b1.reshape(1, _EMB), W2, b2.reshape(1, _EMB),
      gamma.reshape(1, _EMB), beta.reshape(1, _EMB))


def kernel(video_ids, categories, tags, durations, timestamps,
           video_table, category_table, tag_table, duration_table, time_table,
           W1, b1, W2, b2, gamma, beta):
    B, L = video_ids.shape
    n_tok = B * L
    ng = n_tok // (_NW * _G)
    dur_buckets = (durations / 300.0 * 100.0).astype(jnp.int32)
    time_buckets = (timestamps % 168).astype(jnp.int32)
    idx_small = (jnp.stack([
        categories.reshape(-1).astype(jnp.int32),
        tags.reshape(-1).astype(jnp.int32),
        dur_buckets.reshape(-1),
        time_buckets.reshape(-1),
    ]) * 2).reshape(4, _NW, ng, _G)
    idx_video = (video_ids.reshape(-1).astype(jnp.int32)
                 * 2).reshape(1, _NW, ng, _G)

    cat2 = _prep_table(category_table, 2048)
    tag2 = _prep_table(tag_table, 4096)
    dur2 = _prep_table(duration_table, 2048)
    tim2 = _prep_table(time_table, 2048)
    c_rows, t_rows, d_rows, ti_rows = _sc_gather(
        idx_small, [cat2, tag2, dur2, tim2], n_tok)
    vid2 = _prep_table(video_table, 8192)
    (v_rows,) = _sc_gather(idx_video, [vid2], n_tok)

    rows128 = [r.reshape(n_tok // 2, 2 * _EMB)
               for r in (v_rows, c_rows, t_rows, d_rows, ti_rows)]
    out = _tc_mlp(rows128, W1, b1, W2, b2, gamma, beta, n_tok // 2, blk=1024)
    return out.reshape(B, L, _EMB)
